# baseline (speedup 1.0000x reference)
import jax
import jax.numpy as jnp
from jax.experimental import pallas as pl
from jax.experimental.pallas import tpu as pltpu

N = 64
B = 128
BB = 16

_W_SHAPES = [(1536, 640), (1280, 512), (1024, 384), (384, 384), (384, 384),
             (768, 512), (1024, 640), (1280, 768), (768, 768), (768, 768)]


def _ring_agg(h):
    hm = jnp.concatenate([h[:, -1:, :], h[:, :-1, :]], axis=1)
    hp = jnp.concatenate([h[:, 1:, :], h[:, :1, :]], axis=1)
    return (hm + hp) * 0.5


def _dot(a, w):
    return jnp.dot(a.astype(jnp.bfloat16), w,
                   preferred_element_type=jnp.float32)


def _gnn_layer(h, W, b):
    F = h.shape[-1]
    agg = _ring_agg(h)
    h2 = h.reshape(BB * N, F)
    a2 = agg.reshape(BB * N, F)
    y = _dot(h2, W[:F]) + _dot(a2, W[F:]) + b
    return jnp.maximum(y, 0.0).reshape(BB, N, -1)


def _body(x_ref, Wg0, bg0, Wg1, bg1, Wg2, bg2, Wmu, bmu, Wls, bls,
          Wd0, bd0, Wd1, bd1, Wd2, bd2, Wom, bom, Wos, bos,
          epsz_ref, epso_ref, out_ref,
          Sg0, Sg1, Sg2, Smu, Sls, Sd0, Sd1, Sd2, Som, Sos):
    w_refs = (Wg0, Wg1, Wg2, Wmu, Wls, Wd0, Wd1, Wd2, Wom, Wos)
    s_refs = (Sg0, Sg1, Sg2, Smu, Sls, Sd0, Sd1, Sd2, Som, Sos)

    @pl.when(pl.program_id(0) == 0)
    def _cast_weights():
        for w, s in zip(w_refs, s_refs):
            s[...] = w[...].astype(jnp.bfloat16)

    h = x_ref[...]
    h = _gnn_layer(h, Sg0[...], bg0[...])
    h = _gnn_layer(h, Sg1[...], bg1[...])
    h = _gnn_layer(h, Sg2[...], bg2[...])
    h2 = h.reshape(BB * N, 384)
    mu = _dot(h2, Smu[...]) + bmu[...]
    logvar = _dot(h2, Sls[...]) + bls[...]
    z2 = mu + jnp.exp(0.5 * logvar) * epsz_ref[...].reshape(BB * N, 384)
    z = z2.reshape(BB, N, 384)
    d = _gnn_layer(z, Sd0[...], bd0[...])
    d = _gnn_layer(d, Sd1[...], bd1[...])
    d = _gnn_layer(d, Sd2[...], bd2[...])
    d2 = d.reshape(BB * N, 768)
    out_mu = _dot(d2, Som[...]) + bom[...]
    out_sig = jax.nn.softplus(_dot(d2, Sos[...]) + bos[...])
    out = jnp.exp(out_mu + out_sig * epso_ref[...].reshape(BB * N, 768))
    out_ref[...] = out.reshape(BB, N, 768)


def _w_spec(shape):
    return pl.BlockSpec(shape, lambda i: (0,) * len(shape))


def kernel(x, Wg0, bg0, Wg1, bg1, Wg2, bg2, Wmu, bmu, Wls, bls,
           Wd0, bd0, Wd1, bd1, Wd2, bd2, Wom, bom, Wos, bos,
           eps_z, eps_out):
    biases = [b.reshape(1, -1) for b in (bg0, bg1, bg2, bmu, bls, bd0, bd1, bd2, bom, bos)]
    bg0, bg1, bg2, bmu, bls, bd0, bd1, bd2, bom, bos = biases
    weights = (Wg0, bg0, Wg1, bg1, Wg2, bg2, Wmu, bmu, Wls, bls,
               Wd0, bd0, Wd1, bd1, Wd2, bd2, Wom, bom, Wos, bos)
    grid = (B // BB,)
    batch_spec = lambda f: pl.BlockSpec((BB, N, f), lambda i: (i, 0, 0))
    in_specs = [batch_spec(768)]
    in_specs += [_w_spec(w.shape) for w in weights]
    in_specs += [batch_spec(384), batch_spec(768)]
    scratch_shapes = [pltpu.VMEM(s, jnp.bfloat16) for s in _W_SHAPES]
    return pl.pallas_call(
        _body,
        grid=grid,
        in_specs=in_specs,
        out_specs=batch_spec(768),
        out_shape=jax.ShapeDtypeStruct((B, N, 768), jnp.float32),
        scratch_shapes=scratch_shapes,
        compiler_params=pltpu.CompilerParams(
            dimension_semantics=("arbitrary",),
            vmem_limit_bytes=100 * 1024 * 1024,
        ),
    )(x, *weights, eps_z, eps_out)


# speedup vs baseline: 1.0211x; 1.0211x over previous
import jax
import jax.numpy as jnp
from jax.experimental import pallas as pl
from jax.experimental.pallas import tpu as pltpu

N = 64
B = 128
BB = 16

_W_INFO = [((1536, 640), True), ((1280, 512), True), ((1024, 384), True),
           ((384, 384), False), ((384, 384), False),
           ((768, 512), True), ((1024, 640), True), ((1280, 768), True),
           ((768, 768), False), ((768, 768), False)]


def _ring_sum(h):
    hm = jnp.concatenate([h[:, -1:, :], h[:, :-1, :]], axis=1)
    hp = jnp.concatenate([h[:, 1:, :], h[:, :1, :]], axis=1)
    return hm + hp


def _dot(a, w):
    return jnp.dot(a, w, preferred_element_type=jnp.float32)


def _gnn_layer(h, W, b):
    F = h.shape[-1]
    agg = _ring_sum(h)
    h2 = h.reshape(BB * N, F)
    a2 = agg.reshape(BB * N, F)
    y = _dot(h2, W[:F]) + _dot(a2, W[F:]) + b
    return jnp.maximum(y, 0.0).astype(jnp.bfloat16).reshape(BB, N, -1)


def _body(x_ref, Wg0, bg0, Wg1, bg1, Wg2, bg2, Wmu, bmu, Wls, bls,
          Wd0, bd0, Wd1, bd1, Wd2, bd2, Wom, bom, Wos, bos,
          epsz_ref, epso_ref, out_ref,
          Sg0, Sg1, Sg2, Smu, Sls, Sd0, Sd1, Sd2, Som, Sos):
    w_refs = (Wg0, Wg1, Wg2, Wmu, Wls, Wd0, Wd1, Wd2, Wom, Wos)
    s_refs = (Sg0, Sg1, Sg2, Smu, Sls, Sd0, Sd1, Sd2, Som, Sos)

    @pl.when(pl.program_id(0) == 0)
    def _cast_weights():
        for w, s, (shape, is_gnn) in zip(w_refs, s_refs, _W_INFO):
            if is_gnn:
                F = shape[0] // 2
                s[:F, :] = w[:F, :].astype(jnp.bfloat16)
                s[F:, :] = (w[F:, :] * 0.5).astype(jnp.bfloat16)
            else:
                s[...] = w[...].astype(jnp.bfloat16)

    h = x_ref[...].astype(jnp.bfloat16)
    h = _gnn_layer(h, Sg0[...], bg0[...])
    h = _gnn_layer(h, Sg1[...], bg1[...])
    h = _gnn_layer(h, Sg2[...], bg2[...])
    h2 = h.reshape(BB * N, 384)
    mu = _dot(h2, Smu[...]) + bmu[...]
    logvar = _dot(h2, Sls[...]) + bls[...]
    z2 = mu + jnp.exp(0.5 * logvar) * epsz_ref[...].reshape(BB * N, 384)
    z = z2.astype(jnp.bfloat16).reshape(BB, N, 384)
    d = _gnn_layer(z, Sd0[...], bd0[...])
    d = _gnn_layer(d, Sd1[...], bd1[...])
    d = _gnn_layer(d, Sd2[...], bd2[...])
    d2 = d.reshape(BB * N, 768)
    out_mu = _dot(d2, Som[...]) + bom[...]
    out_sig = jax.nn.softplus(_dot(d2, Sos[...]) + bos[...])
    out = jnp.exp(out_mu + out_sig * epso_ref[...].reshape(BB * N, 768))
    out_ref[...] = out.reshape(BB, N, 768)


def _w_spec(shape):
    return pl.BlockSpec(shape, lambda i: (0,) * len(shape))


def kernel(x, Wg0, bg0, Wg1, bg1, Wg2, bg2, Wmu, bmu, Wls, bls,
           Wd0, bd0, Wd1, bd1, Wd2, bd2, Wom, bom, Wos, bos,
           eps_z, eps_out):
    biases = [b.reshape(1, -1) for b in (bg0, bg1, bg2, bmu, bls, bd0, bd1, bd2, bom, bos)]
    bg0, bg1, bg2, bmu, bls, bd0, bd1, bd2, bom, bos = biases
    weights = (Wg0, bg0, Wg1, bg1, Wg2, bg2, Wmu, bmu, Wls, bls,
               Wd0, bd0, Wd1, bd1, Wd2, bd2, Wom, bom, Wos, bos)
    grid = (B // BB,)
    batch_spec = lambda f: pl.BlockSpec((BB, N, f), lambda i: (i, 0, 0))
    in_specs = [batch_spec(768)]
    in_specs += [_w_spec(w.shape) for w in weights]
    in_specs += [batch_spec(384), batch_spec(768)]
    scratch_shapes = [pltpu.VMEM(s, jnp.bfloat16) for s, _ in _W_INFO]
    return pl.pallas_call(
        _body,
        grid=grid,
        in_specs=in_specs,
        out_specs=batch_spec(768),
        out_shape=jax.ShapeDtypeStruct((B, N, 768), jnp.float32),
        scratch_shapes=scratch_shapes,
        compiler_params=pltpu.CompilerParams(
            dimension_semantics=("arbitrary",),
            vmem_limit_bytes=100 * 1024 * 1024,
        ),
    )(x, *weights, eps_z, eps_out)


# speedup vs baseline: 1.1095x; 1.0866x over previous
import jax
import jax.numpy as jnp
from jax.experimental import pallas as pl
from jax.experimental.pallas import tpu as pltpu

N = 64
B = 128
BB = 16

_W_INFO = [((1536, 640), True), ((1280, 512), True), ((1024, 384), True),
           ((384, 384), False), ((384, 384), False),
           ((768, 512), True), ((1024, 640), True), ((1280, 768), True),
           ((768, 768), False), ((768, 768), False)]


def _ring_sum(h):
    hm = jnp.concatenate([h[:, -1:, :], h[:, :-1, :]], axis=1)
    hp = jnp.concatenate([h[:, 1:, :], h[:, :1, :]], axis=1)
    return hm + hp


def _dot(a, w):
    return jnp.dot(a, w, preferred_element_type=jnp.float32)


def _gnn_layer(h, W, b):
    bb, _, F = h.shape
    agg = _ring_sum(h)
    h2 = h.reshape(bb * N, F)
    a2 = agg.reshape(bb * N, F)
    y = _dot(h2, W[:F]) + _dot(a2, W[F:]) + b
    return jnp.maximum(y, 0.0).astype(jnp.bfloat16).reshape(bb, N, -1)


def _body(x_ref, Wg0, bg0, Wg1, bg1, Wg2, bg2, Wmu, bmu, Wls, bls,
          Wd0, bd0, Wd1, bd1, Wd2, bd2, Wom, bom, Wos, bos,
          epsz_ref, epso_ref, out_ref,
          Sg0, Sg1, Sg2, Smu, Sls, Sd0, Sd1, Sd2, Som, Sos):
    w_refs = (Wg0, Wg1, Wg2, Wmu, Wls, Wd0, Wd1, Wd2, Wom, Wos)
    s_refs = (Sg0, Sg1, Sg2, Smu, Sls, Sd0, Sd1, Sd2, Som, Sos)

    @pl.when(pl.program_id(0) == 0)
    def _cast_weights():
        for w, s, (shape, is_gnn) in zip(w_refs, s_refs, _W_INFO):
            if is_gnn:
                F = shape[0] // 2
                s[:F, :] = w[:F, :].astype(jnp.bfloat16)
                s[F:, :] = (w[F:, :] * 0.5).astype(jnp.bfloat16)
            else:
                s[...] = w[...].astype(jnp.bfloat16)

    HB = BB // 2
    M = HB * N
    hs = [x_ref[0:HB].astype(jnp.bfloat16), x_ref[HB:BB].astype(jnp.bfloat16)]
    for S, bias in ((Sg0, bg0), (Sg1, bg1), (Sg2, bg2)):
        W, b = S[...], bias[...]
        hs = [_gnn_layer(h, W, b) for h in hs]
    Wmu_, Wls_ = Smu[...], Sls[...]
    bmu_, bls_ = bmu[...], bls[...]
    zs = []
    for c, h in enumerate(hs):
        h2 = h.reshape(M, 384)
        mu = _dot(h2, Wmu_) + bmu_
        logvar = _dot(h2, Wls_) + bls_
        eps = epsz_ref[c * HB:(c + 1) * HB].reshape(M, 384)
        z2 = mu + jnp.exp(0.5 * logvar) * eps
        zs.append(z2.astype(jnp.bfloat16).reshape(HB, N, 384))
    for S, bias in ((Sd0, bd0), (Sd1, bd1), (Sd2, bd2)):
        W, b = S[...], bias[...]
        zs = [_gnn_layer(z, W, b) for z in zs]
    Wom_, Wos_ = Som[...], Sos[...]
    bom_, bos_ = bom[...], bos[...]
    for c, d in enumerate(zs):
        d2 = d.reshape(M, 768)
        out_mu = _dot(d2, Wom_) + bom_
        out_sig = jax.nn.softplus(_dot(d2, Wos_) + bos_)
        eps = epso_ref[c * HB:(c + 1) * HB].reshape(M, 768)
        out = jnp.exp(out_mu + out_sig * eps)
        out_ref[c * HB:(c + 1) * HB] = out.reshape(HB, N, 768)


def _w_spec(shape):
    return pl.BlockSpec(shape, lambda i: (0,) * len(shape))


def kernel(x, Wg0, bg0, Wg1, bg1, Wg2, bg2, Wmu, bmu, Wls, bls,
           Wd0, bd0, Wd1, bd1, Wd2, bd2, Wom, bom, Wos, bos,
           eps_z, eps_out):
    biases = [b.reshape(1, -1) for b in (bg0, bg1, bg2, bmu, bls, bd0, bd1, bd2, bom, bos)]
    bg0, bg1, bg2, bmu, bls, bd0, bd1, bd2, bom, bos = biases
    weights = (Wg0, bg0, Wg1, bg1, Wg2, bg2, Wmu, bmu, Wls, bls,
               Wd0, bd0, Wd1, bd1, Wd2, bd2, Wom, bom, Wos, bos)
    grid = (B // BB,)
    batch_spec = lambda f: pl.BlockSpec((BB, N, f), lambda i: (i, 0, 0))
    in_specs = [batch_spec(768)]
    in_specs += [_w_spec(w.shape) for w in weights]
    in_specs += [batch_spec(384), batch_spec(768)]
    scratch_shapes = [pltpu.VMEM(s, jnp.bfloat16) for s, _ in _W_INFO]
    return pl.pallas_call(
        _body,
        grid=grid,
        in_specs=in_specs,
        out_specs=batch_spec(768),
        out_shape=jax.ShapeDtypeStruct((B, N, 768), jnp.float32),
        scratch_shapes=scratch_shapes,
        compiler_params=pltpu.CompilerParams(
            dimension_semantics=("arbitrary",),
            vmem_limit_bytes=100 * 1024 * 1024,
        ),
    )(x, *weights, eps_z, eps_out)


# speedup vs baseline: 1.1553x; 1.0413x over previous
import jax
import jax.numpy as jnp
from jax.experimental import pallas as pl
from jax.experimental.pallas import tpu as pltpu

N = 64
B = 128
BB = 16

_W_INFO = [((1536, 640), True), ((1280, 512), True), ((1024, 384), True),
           ((384, 384), False), ((384, 384), False),
           ((768, 512), True), ((1024, 640), True), ((1280, 768), True),
           ((768, 768), False), ((768, 768), False)]


def _ring_sum(h):
    hm = jnp.concatenate([h[:, -1:, :], h[:, :-1, :]], axis=1)
    hp = jnp.concatenate([h[:, 1:, :], h[:, :1, :]], axis=1)
    return hm + hp


def _dot(a, w):
    return jnp.dot(a, w, preferred_element_type=jnp.float32)


def _gnn_layer(h, W, b):
    bb, _, F = h.shape
    agg = _ring_sum(h)
    h2 = h.reshape(bb * N, F)
    a2 = agg.reshape(bb * N, F)
    y = _dot(h2, W[:F]) + _dot(a2, W[F:]) + b
    return jnp.maximum(y, 0.0).astype(jnp.bfloat16).reshape(bb, N, -1)


def _body(x_ref, Wg0, bg0, Wg1, bg1, Wg2, bg2, Wmu, bmu, Wls, bls,
          Wd0, bd0, Wd1, bd1, Wd2, bd2, Wom, bom, Wos, bos,
          epsz_ref, epso_ref, out_ref,
          Sg0, Sg1, Sg2, Smu, Sls, Sd0, Sd1, Sd2, Som, Sos):
    w_refs = (Wg0, Wg1, Wg2, Wmu, Wls, Wd0, Wd1, Wd2, Wom, Wos)
    s_refs = (Sg0, Sg1, Sg2, Smu, Sls, Sd0, Sd1, Sd2, Som, Sos)

    @pl.when(pl.program_id(0) == 0)
    def _cast_weights():
        for w, s, (shape, is_gnn) in zip(w_refs, s_refs, _W_INFO):
            if is_gnn:
                F = shape[0] // 2
                s[:F, :] = w[:F, :].astype(jnp.bfloat16)
                s[F:, :] = (w[F:, :] * 0.5).astype(jnp.bfloat16)
            else:
                s[...] = w[...].astype(jnp.bfloat16)

    NC = 4
    HB = BB // NC
    M = HB * N
    hs = [x_ref[c * HB:(c + 1) * HB].astype(jnp.bfloat16) for c in range(NC)]
    for S, bias in ((Sg0, bg0), (Sg1, bg1), (Sg2, bg2)):
        W, b = S[...], bias[...]
        hs = [_gnn_layer(h, W, b) for h in hs]
    Wmu_, Wls_ = Smu[...], Sls[...]
    bmu_, bls_ = bmu[...], bls[...]
    zs = []
    for c, h in enumerate(hs):
        h2 = h.reshape(M, 384)
        mu = _dot(h2, Wmu_) + bmu_
        logvar = _dot(h2, Wls_) + bls_
        eps = epsz_ref[c * HB:(c + 1) * HB].reshape(M, 384)
        z2 = mu + jnp.exp(0.5 * logvar) * eps
        zs.append(z2.astype(jnp.bfloat16).reshape(HB, N, 384))
    for S, bias in ((Sd0, bd0), (Sd1, bd1), (Sd2, bd2)):
        W, b = S[...], bias[...]
        zs = [_gnn_layer(z, W, b) for z in zs]
    Wom_, Wos_ = Som[...], Sos[...]
    bom_, bos_ = bom[...], bos[...]
    for c, d in enumerate(zs):
        d2 = d.reshape(M, 768)
        out_mu = _dot(d2, Wom_) + bom_
        out_sig = jax.nn.softplus(_dot(d2, Wos_) + bos_)
        eps = epso_ref[c * HB:(c + 1) * HB].reshape(M, 768)
        out = jnp.exp(out_mu + out_sig * eps)
        out_ref[c * HB:(c + 1) * HB] = out.reshape(HB, N, 768)


def _w_spec(shape):
    return pl.BlockSpec(shape, lambda i: (0,) * len(shape))


def kernel(x, Wg0, bg0, Wg1, bg1, Wg2, bg2, Wmu, bmu, Wls, bls,
           Wd0, bd0, Wd1, bd1, Wd2, bd2, Wom, bom, Wos, bos,
           eps_z, eps_out):
    biases = [b.reshape(1, -1) for b in (bg0, bg1, bg2, bmu, bls, bd0, bd1, bd2, bom, bos)]
    bg0, bg1, bg2, bmu, bls, bd0, bd1, bd2, bom, bos = biases
    weights = (Wg0, bg0, Wg1, bg1, Wg2, bg2, Wmu, bmu, Wls, bls,
               Wd0, bd0, Wd1, bd1, Wd2, bd2, Wom, bom, Wos, bos)
    grid = (B // BB,)
    batch_spec = lambda f: pl.BlockSpec((BB, N, f), lambda i: (i, 0, 0))
    in_specs = [batch_spec(768)]
    in_specs += [_w_spec(w.shape) for w in weights]
    in_specs += [batch_spec(384), batch_spec(768)]
    scratch_shapes = [pltpu.VMEM(s, jnp.bfloat16) for s, _ in _W_INFO]
    return pl.pallas_call(
        _body,
        grid=grid,
        in_specs=in_specs,
        out_specs=batch_spec(768),
        out_shape=jax.ShapeDtypeStruct((B, N, 768), jnp.float32),
        scratch_shapes=scratch_shapes,
        compiler_params=pltpu.CompilerParams(
            dimension_semantics=("arbitrary",),
            vmem_limit_bytes=100 * 1024 * 1024,
        ),
    )(x, *weights, eps_z, eps_out)
